# Initial kernel scaffold; baseline (speedup 1.0000x reference)
import functools

import jax
import jax.numpy as jnp
from jax.experimental import pallas as pl

N = 4096
D = 128
_BR = 512


def _pre_body(sH_ref, sK_ref, bH_ref, W_ref, g_ref, h0_ref, Xn_ref):
    step = pl.program_id(0)
    g = jnp.tanh(sH_ref[...])

    @pl.when(step == 0)
    def _init():
        g_ref[...] = g
        h0_ref[...] = -sH_ref[...] + bH_ref[...]
        X = sK_ref[...]
        norm = jnp.sqrt(jnp.sum(X * X, axis=1, keepdims=True))
        Xn_ref[...] = X / norm

    Wb = W_ref[...]
    wg = jnp.dot(Wb, g.T, preferred_element_type=jnp.float32)
    gs = g[:, pl.ds(step * _BR, _BR)]
    wtg = jnp.dot(gs, Wb, preferred_element_type=jnp.float32)
    h0_ref[...] += 0.5 * wtg
    h0_ref[:, pl.ds(step * _BR, _BR)] += 0.5 * wg.T


def _pre(state_H, state_K, bias_H, weights_H):
    grid = N // _BR
    return pl.pallas_call(
        _pre_body,
        grid=(grid,),
        in_specs=[
            pl.BlockSpec((1, N), lambda i: (0, 0)),
            pl.BlockSpec((N, D), lambda i: (0, 0)),
            pl.BlockSpec((1, N), lambda i: (0, 0)),
            pl.BlockSpec((_BR, N), lambda i: (i, 0)),
        ],
        out_specs=[
            pl.BlockSpec((1, N), lambda i: (0, 0)),
            pl.BlockSpec((1, N), lambda i: (0, 0)),
            pl.BlockSpec((N, D), lambda i: (0, 0)),
        ],
        out_shape=[
            jax.ShapeDtypeStruct((1, N), jnp.float32),
            jax.ShapeDtypeStruct((1, N), jnp.float32),
            jax.ShapeDtypeStruct((N, D), jnp.float32),
        ],
    )(state_H[None, :], state_K, bias_H[None, :], weights_H)


def _post_body(Xn_ref, fKp_ref, fHe_ref, h0_ref, fH_ref, fK_ref):
    fH_ref[...] = fHe_ref[...] + h0_ref[...]
    X = Xn_ref[...]
    fKp = fKp_ref[...]
    proj = jnp.sum(X * fKp, axis=1, keepdims=True)
    fK_ref[...] = -fKp + X * proj


def _post(Xn, fKp, fHe, h0):
    return pl.pallas_call(
        _post_body,
        out_shape=[
            jax.ShapeDtypeStruct((1, N), jnp.float32),
            jax.ShapeDtypeStruct((N, D), jnp.float32),
        ],
    )(Xn, fKp, fHe[None, :], h0)


def kernel(t, state_H, state_K, ind_K, ind_HK, kappa_K, kappa_H, weights_H, bias_H, weights_HK, weights_K):
    g2, h0, Xn = _pre(state_H, state_K, bias_H, weights_H)
    g = g2[0]

    i0, i1 = ind_HK[:, 0], ind_HK[:, 1]
    Gram = jnp.sum(Xn[i0] * Xn[i1], axis=1)
    fHe = jnp.zeros((N,), jnp.float32)
    fHe = fHe.at[i0].add(Gram * weights_HK[:, 0] * g[i1] / kappa_H)
    fHe = fHe.at[i1].add(Gram * weights_HK[:, 0] * g[i0] / kappa_H)

    k0, k1 = ind_K[:, 0], ind_K[:, 1]
    s = jnp.sum(Xn[k0] * Xn[k1], axis=1)
    dE_ds = jnp.expand_dims(-weights_K * s, 1)
    fKp = jnp.zeros((N, D), jnp.float32)
    fKp = fKp.at[k0].add(dE_ds * Xn[k1])
    fKp = fKp.at[k1].add(dE_ds * Xn[k0])
    G = jnp.expand_dims(g[i0] * g[i1], 1)
    fKp = fKp.at[i0].add(-G * weights_HK * Xn[i1] / kappa_K)
    fKp = fKp.at[i1].add(-G * weights_HK * Xn[i0] / kappa_K)

    fH2, fK = _post(Xn, fKp, fHe, h0)
    return (fH2[0], fK)


# speedup vs baseline: 1.1093x; 1.1093x over previous
import functools

import jax
import jax.numpy as jnp
from jax.experimental import pallas as pl

N = 4096
D = 128
_BR = 512


def _pre_body(sH_ref, sK_ref, bH_ref, W_ref, g_ref, h0_ref, Xn_ref):
    step = pl.program_id(0)
    g = jnp.tanh(sH_ref[...])

    @pl.when(step == 0)
    def _init():
        g_ref[...] = g
        h0_ref[...] = -sH_ref[...] + bH_ref[...]
        X = sK_ref[...]
        norm = jnp.sqrt(jnp.sum(X * X, axis=1, keepdims=True))
        Xn_ref[...] = X / norm

    Wb = W_ref[...]
    wg = jnp.dot(Wb, g.T, preferred_element_type=jnp.float32)
    gs = jnp.tanh(sH_ref[:, pl.ds(step * _BR, _BR)])
    wtg = jnp.dot(gs, Wb, preferred_element_type=jnp.float32)
    h0_ref[...] += 0.5 * wtg
    h0_ref[:, pl.ds(step * _BR, _BR)] += 0.5 * wg.T


def _pre(state_H, state_K, bias_H, weights_H):
    grid = N // _BR
    return pl.pallas_call(
        _pre_body,
        grid=(grid,),
        in_specs=[
            pl.BlockSpec((1, N), lambda i: (0, 0)),
            pl.BlockSpec((N, D), lambda i: (0, 0)),
            pl.BlockSpec((1, N), lambda i: (0, 0)),
            pl.BlockSpec((_BR, N), lambda i: (i, 0)),
        ],
        out_specs=[
            pl.BlockSpec((1, N), lambda i: (0, 0)),
            pl.BlockSpec((1, N), lambda i: (0, 0)),
            pl.BlockSpec((N, D), lambda i: (0, 0)),
        ],
        out_shape=[
            jax.ShapeDtypeStruct((1, N), jnp.float32),
            jax.ShapeDtypeStruct((1, N), jnp.float32),
            jax.ShapeDtypeStruct((N, D), jnp.float32),
        ],
    )(state_H[None, :], state_K, bias_H[None, :], weights_H)


def _post_body(Xn_ref, fKp_ref, fHe_ref, h0_ref, fH_ref, fK_ref):
    fH_ref[...] = fHe_ref[...] + h0_ref[...]
    X = Xn_ref[...]
    fKp = fKp_ref[...]
    proj = jnp.sum(X * fKp, axis=1, keepdims=True)
    fK_ref[...] = -fKp + X * proj


def _post(Xn, fKp, fHe, h0):
    return pl.pallas_call(
        _post_body,
        out_shape=[
            jax.ShapeDtypeStruct((1, N), jnp.float32),
            jax.ShapeDtypeStruct((N, D), jnp.float32),
        ],
    )(Xn, fKp, fHe[None, :], h0)


def kernel(t, state_H, state_K, ind_K, ind_HK, kappa_K, kappa_H, weights_H, bias_H, weights_HK, weights_K):
    g2, h0, Xn = _pre(state_H, state_K, bias_H, weights_H)
    g = g2[0]

    i0, i1 = ind_HK[:, 0], ind_HK[:, 1]
    Gram = jnp.sum(Xn[i0] * Xn[i1], axis=1)
    fHe = jnp.zeros((N,), jnp.float32)
    fHe = fHe.at[i0].add(Gram * weights_HK[:, 0] * g[i1] / kappa_H)
    fHe = fHe.at[i1].add(Gram * weights_HK[:, 0] * g[i0] / kappa_H)

    k0, k1 = ind_K[:, 0], ind_K[:, 1]
    s = jnp.sum(Xn[k0] * Xn[k1], axis=1)
    dE_ds = jnp.expand_dims(-weights_K * s, 1)
    fKp = jnp.zeros((N, D), jnp.float32)
    fKp = fKp.at[k0].add(dE_ds * Xn[k1])
    fKp = fKp.at[k1].add(dE_ds * Xn[k0])
    G = jnp.expand_dims(g[i0] * g[i1], 1)
    fKp = fKp.at[i0].add(-G * weights_HK * Xn[i1] / kappa_K)
    fKp = fKp.at[i1].add(-G * weights_HK * Xn[i0] / kappa_K)

    fH2, fK = _post(Xn, fKp, fHe, h0)
    return (fH2[0], fK)


# speedup vs baseline: 7.5245x; 6.7832x over previous
import functools

import jax
import jax.numpy as jnp
from jax import lax
from jax.experimental import pallas as pl
from jax.experimental.pallas import tpu as pltpu
from jax.experimental.pallas import tpu_sc as plsc

N = 4096
D = 128
EK = 262144
EHK = 131072
NC = 2
NS = 16
NW = NC * NS
B = 128
CK = EK // NW // B
CHK = EHK // NW // B
RPT = N // NS

_BR = 512


def _pre_body(sH_ref, sK_ref, bH_ref, W_ref, g_ref, h0_ref, Xn_ref):
    step = pl.program_id(0)
    g = jnp.tanh(sH_ref[...])

    @pl.when(step == 0)
    def _init():
        g_ref[...] = g
        h0_ref[...] = -sH_ref[...] + bH_ref[...]
        X = sK_ref[...]
        norm = jnp.sqrt(jnp.sum(X * X, axis=1, keepdims=True))
        Xn_ref[...] = X / norm

    Wb = W_ref[...]
    wg = jnp.dot(Wb, g.T, preferred_element_type=jnp.float32)
    gs = jnp.tanh(sH_ref[:, pl.ds(step * _BR, _BR)])
    wtg = jnp.dot(gs, Wb, preferred_element_type=jnp.float32)
    h0_ref[...] += 0.5 * wtg
    h0_ref[:, pl.ds(step * _BR, _BR)] += 0.5 * wg.T


def _pre(state_H, state_K, bias_H, weights_H):
    return pl.pallas_call(
        _pre_body,
        grid=(N // _BR,),
        in_specs=[
            pl.BlockSpec((1, N), lambda i: (0, 0)),
            pl.BlockSpec((N, D), lambda i: (0, 0)),
            pl.BlockSpec((1, N), lambda i: (0, 0)),
            pl.BlockSpec((_BR, N), lambda i: (i, 0)),
        ],
        out_specs=[
            pl.BlockSpec((1, N), lambda i: (0, 0)),
            pl.BlockSpec((1, N), lambda i: (0, 0)),
            pl.BlockSpec((N, D), lambda i: (0, 0)),
        ],
        out_shape=[
            jax.ShapeDtypeStruct((1, N), jnp.float32),
            jax.ShapeDtypeStruct((1, N), jnp.float32),
            jax.ShapeDtypeStruct((N, D), jnp.float32),
        ],
    )(state_H[None, :], state_K, bias_H[None, :], weights_H)


_mesh = plsc.VectorSubcoreMesh(core_axis_name="c", subcore_axis_name="s")


def _splat(ref, e):
    return plsc.load_gather(ref, [jnp.full((16,), e, jnp.int32)])


@functools.partial(
    pl.kernel,
    out_type=[
        jax.ShapeDtypeStruct((NC, N, D), jnp.float32),
        jax.ShapeDtypeStruct((NW, N), jnp.float32),
    ],
    mesh=_mesh,
    compiler_params=pltpu.CompilerParams(needs_layout_passes=False),
    scratch_types=[
        pltpu.VMEM((B,), jnp.int32),
        pltpu.VMEM((B,), jnp.int32),
        pltpu.VMEM((B,), jnp.float32),
        pltpu.VMEM((B,), jnp.float32),
        pltpu.VMEM((B, D), jnp.float32),
        pltpu.VMEM((B, D), jnp.float32),
        pltpu.VMEM((B, D), jnp.float32),
        pltpu.VMEM((B, D), jnp.float32),
        pltpu.VMEM((N,), jnp.float32),
        pltpu.VMEM((N,), jnp.float32),
        pltpu.VMEM_SHARED((N, D), jnp.float32),
        pltpu.SemaphoreType.DMA,
        pltpu.SemaphoreType.DMA,
    ],
)
def _edges(xn, g, ik0, ik1, wk, ih0, ih1, whh, whk, fko, fho,
           idx_i, idx_j, wbuf, w2buf, xi, xj, yi, yj, gall, fh, fk_acc,
           sem1, sem2):
    cid = lax.axis_index("c")
    sid = lax.axis_index("s")
    wid = cid * NS + sid
    z16 = jnp.zeros((16,), jnp.float32)
    lane0 = jnp.arange(16, dtype=jnp.int32) == 0

    def _zrow(r, _):
        for k in range(D // 16):
            yi[r, pl.ds(k * 16, 16)] = z16
        return 0

    lax.fori_loop(0, B, _zrow, 0)

    def _zfh(i, _):
        fh[pl.ds(i * 16, 16)] = z16
        return 0

    lax.fori_loop(0, N // 16, _zfh, 0)

    pltpu.sync_copy(yi, fk_acc.at[pl.ds(sid * RPT, B)])
    pltpu.sync_copy(yi, fk_acc.at[pl.ds(sid * RPT + B, B)])
    pltpu.sync_copy(g, gall)
    plsc.subcore_barrier()

    def _k_chunk(c, _):
        base = wid * (CK * B) + c * B
        pltpu.sync_copy(ik0.at[pl.ds(base, B)], idx_i)
        pltpu.sync_copy(ik1.at[pl.ds(base, B)], idx_j)
        pltpu.sync_copy(wk.at[pl.ds(base, B)], wbuf)
        pltpu.async_copy(xn.at[idx_i], xi, sem1).wait()
        pltpu.async_copy(xn.at[idx_j], xj, sem2).wait()

        def _edge(e, _):
            xiv = [xi[e, pl.ds(k * 16, 16)] for k in range(D // 16)]
            xjv = [xj[e, pl.ds(k * 16, 16)] for k in range(D // 16)]
            acc = xiv[0] * xjv[0]
            for k in range(1, D // 16):
                acc = acc + xiv[k] * xjv[k]
            sv = jnp.broadcast_to(jnp.sum(acc), (16,))
            qv = -_splat(wbuf, e) * sv
            for k in range(D // 16):
                yi[e, pl.ds(k * 16, 16)] = qv * xjv[k]
                yj[e, pl.ds(k * 16, 16)] = qv * xiv[k]
            return 0

        lax.fori_loop(0, B, _edge, 0)
        pltpu.sync_copy(yi, fk_acc.at[idx_i], add=True)
        pltpu.sync_copy(yj, fk_acc.at[idx_j], add=True)
        return 0

    lax.fori_loop(0, CK, _k_chunk, 0)

    def _hk_chunk(c, _):
        base = wid * (CHK * B) + c * B
        pltpu.sync_copy(ih0.at[pl.ds(base, B)], idx_i)
        pltpu.sync_copy(ih1.at[pl.ds(base, B)], idx_j)
        pltpu.sync_copy(whk.at[pl.ds(base, B)], wbuf)
        pltpu.sync_copy(whh.at[pl.ds(base, B)], w2buf)
        pltpu.async_copy(xn.at[idx_i], xi, sem1).wait()
        pltpu.async_copy(xn.at[idx_j], xj, sem2).wait()

        def _edge(e, _):
            xiv = [xi[e, pl.ds(k * 16, 16)] for k in range(D // 16)]
            xjv = [xj[e, pl.ds(k * 16, 16)] for k in range(D // 16)]
            acc = xiv[0] * xjv[0]
            for k in range(1, D // 16):
                acc = acc + xiv[k] * xjv[k]
            sv = jnp.broadcast_to(jnp.sum(acc), (16,))
            iiv = _splat(idx_i, e)
            jjv = _splat(idx_j, e)
            giv = plsc.load_gather(gall, [iiv])
            gjv = plsc.load_gather(gall, [jjv])
            cv = _splat(w2buf, e) * sv
            plsc.addupdate_scatter(fh, [iiv], cv * gjv, mask=lane0)
            plsc.addupdate_scatter(fh, [jjv], cv * giv, mask=lane0)
            qv = -(giv * gjv * _splat(wbuf, e))
            for k in range(D // 16):
                yi[e, pl.ds(k * 16, 16)] = qv * xjv[k]
                yj[e, pl.ds(k * 16, 16)] = qv * xiv[k]
            return 0

        lax.fori_loop(0, B, _edge, 0)
        pltpu.sync_copy(yi, fk_acc.at[idx_i], add=True)
        pltpu.sync_copy(yj, fk_acc.at[idx_j], add=True)
        return 0

    lax.fori_loop(0, CHK, _hk_chunk, 0)
    plsc.subcore_barrier()

    pltpu.sync_copy(fk_acc.at[pl.ds(sid * RPT, RPT)],
                    fko.at[cid, pl.ds(sid * RPT, RPT)])
    pltpu.sync_copy(fh, fho.at[wid])


def _post_body(Xn_ref, fkp_ref, fhp_ref, h0_ref, fH_ref, fK_ref):
    fH_ref[...] = h0_ref[...] + jnp.sum(fhp_ref[...], axis=0, keepdims=True)
    X = Xn_ref[...]
    fKp = fkp_ref[0] + fkp_ref[1]
    proj = jnp.sum(X * fKp, axis=1, keepdims=True)
    fK_ref[...] = -fKp + X * proj


def _post(Xn, fkp, fhp, h0):
    return pl.pallas_call(
        _post_body,
        out_shape=[
            jax.ShapeDtypeStruct((1, N), jnp.float32),
            jax.ShapeDtypeStruct((N, D), jnp.float32),
        ],
    )(Xn, fkp, fhp, h0)


def kernel(t, state_H, state_K, ind_K, ind_HK, kappa_K, kappa_H, weights_H, bias_H, weights_HK, weights_K):
    g2, h0, Xn = _pre(state_H, state_K, bias_H, weights_H)
    g = g2.reshape(N)
    whh = weights_HK[:, 0] / kappa_H
    whk = weights_HK[:, 0] / kappa_K
    fkp, fhp = _edges(Xn, g, ind_K[:, 0], ind_K[:, 1], weights_K,
                      ind_HK[:, 0], ind_HK[:, 1], whh, whk)
    fH2, fK = _post(Xn, fkp, fhp, h0)
    return (fH2.reshape(N), fK)


# speedup vs baseline: 16.3570x; 2.1738x over previous
import functools

import jax
import jax.numpy as jnp
from jax import lax
from jax.experimental import pallas as pl
from jax.experimental.pallas import tpu as pltpu
from jax.experimental.pallas import tpu_sc as plsc

N = 4096
D = 128
EK = 262144
EHK = 131072
NC = 2
NS = 16
NW = NC * NS
B = 128
CK = EK // NW // B
CHK = EHK // NW // B
RPT = N // NS

_BR = 512


def _pre_body(sH_ref, sK_ref, bH_ref, W_ref, g_ref, h0_ref, Xn_ref):
    step = pl.program_id(0)
    g = jnp.tanh(sH_ref[...])

    @pl.when(step == 0)
    def _init():
        g_ref[...] = g
        h0_ref[...] = -sH_ref[...] + bH_ref[...]
        X = sK_ref[...]
        norm = jnp.sqrt(jnp.sum(X * X, axis=1, keepdims=True))
        Xn_ref[...] = X / norm

    Wb = W_ref[...]
    wg = jnp.dot(Wb, g.T, preferred_element_type=jnp.float32)
    gs = jnp.tanh(sH_ref[:, pl.ds(step * _BR, _BR)])
    wtg = jnp.dot(gs, Wb, preferred_element_type=jnp.float32)
    h0_ref[...] += 0.5 * wtg
    h0_ref[:, pl.ds(step * _BR, _BR)] += 0.5 * wg.T


def _pre(state_H, state_K, bias_H, weights_H):
    return pl.pallas_call(
        _pre_body,
        grid=(N // _BR,),
        in_specs=[
            pl.BlockSpec((1, N), lambda i: (0, 0)),
            pl.BlockSpec((N, D), lambda i: (0, 0)),
            pl.BlockSpec((1, N), lambda i: (0, 0)),
            pl.BlockSpec((_BR, N), lambda i: (i, 0)),
        ],
        out_specs=[
            pl.BlockSpec((1, N), lambda i: (0, 0)),
            pl.BlockSpec((1, N), lambda i: (0, 0)),
            pl.BlockSpec((N, D), lambda i: (0, 0)),
        ],
        out_shape=[
            jax.ShapeDtypeStruct((1, N), jnp.float32),
            jax.ShapeDtypeStruct((1, N), jnp.float32),
            jax.ShapeDtypeStruct((N, D), jnp.float32),
        ],
    )(state_H[None, :], state_K, bias_H[None, :], weights_H)


_mesh = plsc.VectorSubcoreMesh(core_axis_name="c", subcore_axis_name="s")


def _splat(ref1d, e):
    return plsc.load_gather(ref1d, [jnp.full((16,), e, jnp.int32)])


@functools.partial(
    pl.kernel,
    out_type=[
        jax.ShapeDtypeStruct((NC, N, D), jnp.float32),
        jax.ShapeDtypeStruct((NW, N), jnp.float32),
    ],
    mesh=_mesh,
    compiler_params=pltpu.CompilerParams(needs_layout_passes=False),
    scratch_types=[
        pltpu.VMEM((B,), jnp.int32),
        pltpu.VMEM((B,), jnp.int32),
        pltpu.VMEM((B,), jnp.int32),
        pltpu.VMEM((B,), jnp.int32),
        pltpu.VMEM((B,), jnp.int32),
        pltpu.VMEM((B,), jnp.int32),
        pltpu.VMEM((B,), jnp.int32),
        pltpu.VMEM((B,), jnp.int32),
        pltpu.VMEM((B,), jnp.float32),
        pltpu.VMEM((B,), jnp.float32),
        pltpu.VMEM((B,), jnp.float32),
        pltpu.VMEM((B,), jnp.float32),
        pltpu.VMEM((B, D), jnp.float32),
        pltpu.VMEM((B, D), jnp.float32),
        pltpu.VMEM((B, D), jnp.float32),
        pltpu.VMEM((B, D), jnp.float32),
        pltpu.VMEM((N,), jnp.float32),
        pltpu.VMEM((N,), jnp.float32),
        pltpu.VMEM_SHARED((N, D), jnp.float32),
        pltpu.SemaphoreType.DMA,
        pltpu.SemaphoreType.DMA,
        pltpu.SemaphoreType.DMA,
        pltpu.SemaphoreType.DMA,
        pltpu.SemaphoreType.DMA,
        pltpu.SemaphoreType.DMA,
    ],
)
def _edges(xn, g, ik0, ik1, wk, ih0, ih1, whk, whh, fko, fho,
           gi0, gj0, gi1, gj1, si0, sj0, si1, sj1,
           wa0, wa1, wb0, wb1,
           xi0, xj0, xi1, xj1, gall, fh, fk_acc,
           sg0, sg1, ss0, ss1, si0s, si1s):
    cid = lax.axis_index("c")
    sid = lax.axis_index("s")
    wid = cid * NS + sid
    z16 = jnp.zeros((16,), jnp.float32)
    lane0 = jnp.arange(16, dtype=jnp.int32) == 0
    GI = (gi0, gi1)
    GJ = (gj0, gj1)
    SI = (si0, si1)
    SJ = (sj0, sj1)
    WA = (wa0, wa1)
    WB = (wb0, wb1)
    XI = (xi0, xi1)
    XJ = (xj0, xj1)
    SG = (sg0, sg1)
    SS = (ss0, ss1)
    SM = (si0s, si1s)

    pltpu.sync_copy(g, gall)

    @plsc.parallel_loop(0, B)
    def _zrow(r):
        for k in range(D // 16):
            xi0[r, pl.ds(k * 16, 16)] = z16

    @plsc.parallel_loop(0, N // 16)
    def _zfh(i):
        fh[pl.ds(i * 16, 16)] = z16

    for r0 in range(0, RPT, B):
        pltpu.sync_copy(xi0, fk_acc.at[pl.ds(sid * RPT + r0, B)])
    plsc.subcore_barrier()

    def _phase(base0, C, i0_h, i1_h, wa_h, wb_h, hk):
        def _idx_dma(n, b):
            off = base0 + n * B
            pltpu.async_copy(i0_h.at[pl.ds(off, B)], GI[b], SM[b])
            pltpu.async_copy(i1_h.at[pl.ds(off, B)], GJ[b], SM[b])
            pltpu.async_copy(wa_h.at[pl.ds(off, B)], WA[b], SM[b])
            if hk:
                pltpu.async_copy(wb_h.at[pl.ds(off, B)], WB[b], SM[b])

        def _drain_idx(n, b):
            off = base0 + n * B
            pltpu.make_async_copy(i0_h.at[pl.ds(off, B)], GI[b], SM[b]).wait()
            pltpu.make_async_copy(i1_h.at[pl.ds(off, B)], GJ[b], SM[b]).wait()
            pltpu.make_async_copy(wa_h.at[pl.ds(off, B)], WA[b], SM[b]).wait()
            if hk:
                pltpu.make_async_copy(wb_h.at[pl.ds(off, B)], WB[b], SM[b]).wait()

        def _gather(b):
            pltpu.async_copy(xn.at[GI[b]], XI[b], SG[b])
            pltpu.async_copy(xn.at[GJ[b]], XJ[b], SG[b])

        def _drain_gather(b):
            pltpu.make_async_copy(xn.at[GI[b]], XI[b], SG[b]).wait()
            pltpu.make_async_copy(xn.at[GJ[b]], XJ[b], SG[b]).wait()

        def _scatter(b):
            pltpu.async_copy(XJ[b], fk_acc.at[SI[b]], SS[b], add=True)
            pltpu.async_copy(XI[b], fk_acc.at[SJ[b]], SS[b], add=True)

        def _drain_scatter(b):
            pltpu.make_async_copy(XJ[b], fk_acc.at[SI[b]], SS[b]).wait()
            pltpu.make_async_copy(XI[b], fk_acc.at[SJ[b]], SS[b]).wait()

        def _compute(b):
            xi_b, xj_b = XI[b], XJ[b]
            wa_b, wb_b = WA[b], WB[b]
            gi_b, gj_b = GI[b], GJ[b]

            @plsc.parallel_loop(0, B, unroll=4)
            def _edge(e):
                xiv = [xi_b[e, pl.ds(k * 16, 16)] for k in range(D // 16)]
                xjv = [xj_b[e, pl.ds(k * 16, 16)] for k in range(D // 16)]
                acc = xiv[0] * xjv[0]
                for k in range(1, D // 16):
                    acc = acc + xiv[k] * xjv[k]
                sv = jnp.broadcast_to(jnp.sum(acc), (16,))
                if hk:
                    iiv = _splat(gi_b, e)
                    jjv = _splat(gj_b, e)
                    giv = plsc.load_gather(gall, [iiv])
                    gjv = plsc.load_gather(gall, [jjv])
                    cv = _splat(wb_b, e) * sv
                    plsc.addupdate_scatter(fh, [iiv], cv * gjv, mask=lane0)
                    plsc.addupdate_scatter(fh, [jjv], cv * giv, mask=lane0)
                    qv = -(giv * gjv * _splat(wa_b, e))
                else:
                    qv = -_splat(wa_b, e) * sv
                for k in range(D // 16):
                    xj_b[e, pl.ds(k * 16, 16)] = qv * xjv[k]
                    xi_b[e, pl.ds(k * 16, 16)] = qv * xiv[k]

        def _save_idx(b):
            for k in range(B // 16):
                s = pl.ds(k * 16, 16)
                SI[b][s] = GI[b][s]
                SJ[b][s] = GJ[b][s]

        _idx_dma(0, 0)
        _idx_dma(1, 1)
        _drain_idx(0, 0)
        _gather(0)

        @pl.loop(0, C, step=2)
        def _loop(c):
            for b in (0, 1):
                o = 1 - b
                n = c + b
                if b == 0:
                    @pl.when(c > 0)
                    def _():
                        _drain_scatter(o)
                    _drain_idx(n + 1, o)
                    _gather(o)
                else:
                    _drain_scatter(o)

                    @pl.when(c < C - 2)
                    def _():
                        _drain_idx(n + 1, o)
                        _gather(o)
                _drain_gather(b)
                _save_idx(b)
                _compute(b)

                @pl.when(c < C - 2)
                def _():
                    _idx_dma(n + 2, b)
                _scatter(b)

        _drain_scatter((C - 1) % 2)

    _phase(wid * (CK * B), CK, ik0, ik1, wk, wk, False)
    _phase(wid * (CHK * B), CHK, ih0, ih1, whk, whh, True)
    plsc.subcore_barrier()

    pltpu.sync_copy(fk_acc.at[pl.ds(sid * RPT, RPT)],
                    fko.at[cid, pl.ds(sid * RPT, RPT)])
    pltpu.sync_copy(fh, fho.at[wid])


def _post_body(Xn_ref, fkp_ref, fhp_ref, h0_ref, fH_ref, fK_ref):
    fH_ref[...] = h0_ref[...] + jnp.sum(fhp_ref[...], axis=0, keepdims=True)
    X = Xn_ref[...]
    fKp = fkp_ref[0] + fkp_ref[1]
    proj = jnp.sum(X * fKp, axis=1, keepdims=True)
    fK_ref[...] = -fKp + X * proj


def _post(Xn, fkp, fhp, h0):
    return pl.pallas_call(
        _post_body,
        out_shape=[
            jax.ShapeDtypeStruct((1, N), jnp.float32),
            jax.ShapeDtypeStruct((N, D), jnp.float32),
        ],
    )(Xn, fkp, fhp, h0)


def kernel(t, state_H, state_K, ind_K, ind_HK, kappa_K, kappa_H, weights_H, bias_H, weights_HK, weights_K):
    g2, h0, Xn = _pre(state_H, state_K, bias_H, weights_H)
    g = g2.reshape(N)
    whh = weights_HK[:, 0] / kappa_H
    whk = weights_HK[:, 0] / kappa_K
    fkp, fhp = _edges(Xn, g, ind_K[:, 0], ind_K[:, 1], weights_K,
                      ind_HK[:, 0], ind_HK[:, 1], whk, whh)
    fH2, fK = _post(Xn, fkp, fhp, h0)
    return (fH2.reshape(N), fK)


# speedup vs baseline: 19.5591x; 1.1958x over previous
import functools

import jax
import jax.numpy as jnp
from jax import lax
from jax.experimental import pallas as pl
from jax.experimental.pallas import tpu as pltpu
from jax.experimental.pallas import tpu_sc as plsc

N = 4096
D = 128
EK = 262144
EHK = 131072
NC = 2
NS = 16
NW = NC * NS
B = 128
CK = EK // NW // B
CHK = EHK // NW // B
RPT = N // NS

_BR = 512


def _pre_body(sH_ref, sK_ref, bH_ref, W_ref, g_ref, h0_ref, Xn_ref):
    step = pl.program_id(0)
    g = jnp.tanh(sH_ref[...])

    @pl.when(step == 0)
    def _init():
        g_ref[...] = g
        h0_ref[...] = -sH_ref[...] + bH_ref[...]
        X = sK_ref[...]
        norm = jnp.sqrt(jnp.sum(X * X, axis=1, keepdims=True))
        Xn_ref[...] = X / norm

    Wb = W_ref[...]
    wg = jnp.dot(Wb, g.T, preferred_element_type=jnp.float32)
    gs = jnp.tanh(sH_ref[:, pl.ds(step * _BR, _BR)])
    wtg = jnp.dot(gs, Wb, preferred_element_type=jnp.float32)
    h0_ref[...] += 0.5 * wtg
    h0_ref[:, pl.ds(step * _BR, _BR)] += 0.5 * wg.T


def _pre(state_H, state_K, bias_H, weights_H):
    return pl.pallas_call(
        _pre_body,
        grid=(N // _BR,),
        in_specs=[
            pl.BlockSpec((1, N), lambda i: (0, 0)),
            pl.BlockSpec((N, D), lambda i: (0, 0)),
            pl.BlockSpec((1, N), lambda i: (0, 0)),
            pl.BlockSpec((_BR, N), lambda i: (i, 0)),
        ],
        out_specs=[
            pl.BlockSpec((1, N), lambda i: (0, 0)),
            pl.BlockSpec((1, N), lambda i: (0, 0)),
            pl.BlockSpec((N, D), lambda i: (0, 0)),
        ],
        out_shape=[
            jax.ShapeDtypeStruct((1, N), jnp.float32),
            jax.ShapeDtypeStruct((1, N), jnp.float32),
            jax.ShapeDtypeStruct((N, D), jnp.float32),
        ],
    )(state_H[None, :], state_K, bias_H[None, :], weights_H)


_mesh = plsc.VectorSubcoreMesh(core_axis_name="c", subcore_axis_name="s")


def _splat(ref1d, e):
    return plsc.load_gather(ref1d, [jnp.full((16,), e, jnp.int32)])


@functools.partial(
    pl.kernel,
    out_type=[
        jax.ShapeDtypeStruct((NC, N, D), jnp.float32),
        jax.ShapeDtypeStruct((NW, N), jnp.float32),
    ],
    mesh=_mesh,
    compiler_params=pltpu.CompilerParams(needs_layout_passes=False),
    scratch_types=[
        pltpu.VMEM((B,), jnp.int32),
        pltpu.VMEM((B,), jnp.int32),
        pltpu.VMEM((B,), jnp.int32),
        pltpu.VMEM((B,), jnp.int32),
        pltpu.VMEM((B,), jnp.int32),
        pltpu.VMEM((B,), jnp.int32),
        pltpu.VMEM((B,), jnp.int32),
        pltpu.VMEM((B,), jnp.int32),
        pltpu.VMEM((B,), jnp.float32),
        pltpu.VMEM((B,), jnp.float32),
        pltpu.VMEM((B,), jnp.float32),
        pltpu.VMEM((B,), jnp.float32),
        pltpu.VMEM((B, D), jnp.float32),
        pltpu.VMEM((B, D), jnp.float32),
        pltpu.VMEM((B, D), jnp.float32),
        pltpu.VMEM((B, D), jnp.float32),
        pltpu.VMEM((N,), jnp.float32),
        pltpu.VMEM((N,), jnp.float32),
        pltpu.VMEM_SHARED((N, D), jnp.float32),
        pltpu.SemaphoreType.DMA,
        pltpu.SemaphoreType.DMA,
        pltpu.SemaphoreType.DMA,
        pltpu.SemaphoreType.DMA,
        pltpu.SemaphoreType.DMA,
        pltpu.SemaphoreType.DMA,
    ],
)
def _edges(xn, g, ik0, ik1, wk, ih0, ih1, whk, whh, fko, fho,
           gi0, gj0, gi1, gj1, si0, sj0, si1, sj1,
           wa0, wa1, wb0, wb1,
           xi0, xj0, xi1, xj1, gall, fh, fk_acc,
           sg0, sg1, ss0, ss1, si0s, si1s):
    cid = lax.axis_index("c")
    sid = lax.axis_index("s")
    wid = cid * NS + sid
    z16 = jnp.zeros((16,), jnp.float32)
    lane0 = jnp.arange(16, dtype=jnp.int32) == 0
    GI = (gi0, gi1)
    GJ = (gj0, gj1)
    SI = (si0, si1)
    SJ = (sj0, sj1)
    WA = (wa0, wa1)
    WB = (wb0, wb1)
    XI = (xi0, xi1)
    XJ = (xj0, xj1)
    SG = (sg0, sg1)
    SS = (ss0, ss1)
    SM = (si0s, si1s)

    pltpu.sync_copy(g, gall)

    @plsc.parallel_loop(0, B)
    def _zrow(r):
        for k in range(D // 16):
            xi0[r, pl.ds(k * 16, 16)] = z16

    @plsc.parallel_loop(0, N // 16)
    def _zfh(i):
        fh[pl.ds(i * 16, 16)] = z16

    for r0 in range(0, RPT, B):
        pltpu.sync_copy(xi0, fk_acc.at[pl.ds(sid * RPT + r0, B)])
    plsc.subcore_barrier()

    def _phase(base0, C, i0_h, i1_h, wa_h, wb_h, hk):
        def _idx_dma(n, b):
            off = base0 + n * B
            pltpu.async_copy(i0_h.at[pl.ds(off, B)], GI[b], SM[b])
            pltpu.async_copy(i1_h.at[pl.ds(off, B)], GJ[b], SM[b])
            pltpu.async_copy(wa_h.at[pl.ds(off, B)], WA[b], SM[b])
            if hk:
                pltpu.async_copy(wb_h.at[pl.ds(off, B)], WB[b], SM[b])

        def _drain_idx(n, b):
            off = base0 + n * B
            pltpu.make_async_copy(i0_h.at[pl.ds(off, B)], GI[b], SM[b]).wait()
            pltpu.make_async_copy(i1_h.at[pl.ds(off, B)], GJ[b], SM[b]).wait()
            pltpu.make_async_copy(wa_h.at[pl.ds(off, B)], WA[b], SM[b]).wait()
            if hk:
                pltpu.make_async_copy(wb_h.at[pl.ds(off, B)], WB[b], SM[b]).wait()

        def _gather(b):
            pltpu.async_copy(xn.at[GI[b]], XI[b], SG[b])
            pltpu.async_copy(xn.at[GJ[b]], XJ[b], SG[b])

        def _drain_gather(b):
            pltpu.make_async_copy(xn.at[GI[b]], XI[b], SG[b]).wait()
            pltpu.make_async_copy(xn.at[GJ[b]], XJ[b], SG[b]).wait()

        def _scatter(b):
            pltpu.async_copy(XJ[b], fk_acc.at[SI[b]], SS[b], add=True)
            pltpu.async_copy(XI[b], fk_acc.at[SJ[b]], SS[b], add=True)

        def _drain_scatter(b):
            pltpu.make_async_copy(XJ[b], fk_acc.at[SI[b]], SS[b]).wait()
            pltpu.make_async_copy(XI[b], fk_acc.at[SJ[b]], SS[b]).wait()

        def _compute(b):
            xi_b, xj_b = XI[b], XJ[b]
            wa_b, wb_b = WA[b], WB[b]
            gi_b, gj_b = GI[b], GJ[b]

            @plsc.parallel_loop(0, B, unroll=8)
            def _edge(e):
                xiv = [xi_b[e, pl.ds(k * 16, 16)] for k in range(D // 16)]
                xjv = [xj_b[e, pl.ds(k * 16, 16)] for k in range(D // 16)]
                acc = xiv[0] * xjv[0]
                for k in range(1, D // 16):
                    acc = acc + xiv[k] * xjv[k]
                sv = jnp.broadcast_to(jnp.sum(acc), (16,))
                if hk:
                    iiv = _splat(gi_b, e)
                    jjv = _splat(gj_b, e)
                    giv = plsc.load_gather(gall, [iiv])
                    gjv = plsc.load_gather(gall, [jjv])
                    cv = _splat(wb_b, e) * sv
                    plsc.addupdate_scatter(fh, [iiv], cv * gjv, mask=lane0)
                    plsc.addupdate_scatter(fh, [jjv], cv * giv, mask=lane0)
                    qv = -(giv * gjv * _splat(wa_b, e))
                else:
                    qv = -_splat(wa_b, e) * sv
                for k in range(D // 16):
                    xj_b[e, pl.ds(k * 16, 16)] = qv * xjv[k]
                    xi_b[e, pl.ds(k * 16, 16)] = qv * xiv[k]

        def _save_idx(b):
            for k in range(B // 16):
                s = pl.ds(k * 16, 16)
                SI[b][s] = GI[b][s]
                SJ[b][s] = GJ[b][s]

        _idx_dma(0, 0)
        _idx_dma(1, 1)
        _drain_idx(0, 0)
        _gather(0)

        @pl.loop(0, C, step=2)
        def _loop(c):
            for b in (0, 1):
                o = 1 - b
                n = c + b
                if b == 0:
                    @pl.when(c > 0)
                    def _():
                        _drain_scatter(o)
                    _drain_idx(n + 1, o)
                    _gather(o)
                else:
                    _drain_scatter(o)

                    @pl.when(c < C - 2)
                    def _():
                        _drain_idx(n + 1, o)
                        _gather(o)
                _drain_gather(b)
                _save_idx(b)
                _compute(b)

                @pl.when(c < C - 2)
                def _():
                    _idx_dma(n + 2, b)
                _scatter(b)

        _drain_scatter((C - 1) % 2)

    _phase(wid * (CK * B), CK, ik0, ik1, wk, wk, False)
    _phase(wid * (CHK * B), CHK, ih0, ih1, whk, whh, True)
    plsc.subcore_barrier()

    pltpu.sync_copy(fk_acc.at[pl.ds(sid * RPT, RPT)],
                    fko.at[cid, pl.ds(sid * RPT, RPT)])
    pltpu.sync_copy(fh, fho.at[wid])


def _post_body(Xn_ref, fkp_ref, fhp_ref, h0_ref, fH_ref, fK_ref):
    fH_ref[...] = h0_ref[...] + jnp.sum(fhp_ref[...], axis=0, keepdims=True)
    X = Xn_ref[...]
    fKp = fkp_ref[0] + fkp_ref[1]
    proj = jnp.sum(X * fKp, axis=1, keepdims=True)
    fK_ref[...] = -fKp + X * proj


def _post(Xn, fkp, fhp, h0):
    return pl.pallas_call(
        _post_body,
        out_shape=[
            jax.ShapeDtypeStruct((1, N), jnp.float32),
            jax.ShapeDtypeStruct((N, D), jnp.float32),
        ],
    )(Xn, fkp, fhp, h0)


def kernel(t, state_H, state_K, ind_K, ind_HK, kappa_K, kappa_H, weights_H, bias_H, weights_HK, weights_K):
    g2, h0, Xn = _pre(state_H, state_K, bias_H, weights_H)
    g = g2.reshape(N)
    whh = weights_HK[:, 0] / kappa_H
    whk = weights_HK[:, 0] / kappa_K
    fkp, fhp = _edges(Xn, g, ind_K[:, 0], ind_K[:, 1], weights_K,
                      ind_HK[:, 0], ind_HK[:, 1], whk, whh)
    fH2, fK = _post(Xn, fkp, fhp, h0)
    return (fH2.reshape(N), fK)


# speedup vs baseline: 19.7300x; 1.0087x over previous
import functools

import jax
import jax.numpy as jnp
from jax import lax
from jax.experimental import pallas as pl
from jax.experimental.pallas import tpu as pltpu
from jax.experimental.pallas import tpu_sc as plsc

N = 4096
D = 128
EK = 262144
EHK = 131072
NC = 2
NS = 16
NW = NC * NS
B = 128
CK = EK // NW // B
CHK = EHK // NW // B
RPT = N // NS

_BR = 512


def _pre_body(sH_ref, sK_ref, bH_ref, W_ref, g_ref, h0_ref, Xn_ref):
    step = pl.program_id(0)
    g = jnp.tanh(sH_ref[...])

    @pl.when(step == 0)
    def _init():
        g_ref[...] = g
        h0_ref[...] = -sH_ref[...] + bH_ref[...]
        X = sK_ref[...]
        norm = jnp.sqrt(jnp.sum(X * X, axis=1, keepdims=True))
        Xn_ref[...] = X / norm

    Wb = W_ref[...]
    wg = jnp.dot(Wb, g.T, preferred_element_type=jnp.float32)
    gs = jnp.tanh(sH_ref[:, pl.ds(step * _BR, _BR)])
    wtg = jnp.dot(gs, Wb, preferred_element_type=jnp.float32)
    h0_ref[...] += 0.5 * wtg
    h0_ref[:, pl.ds(step * _BR, _BR)] += 0.5 * wg.T


def _pre(state_H, state_K, bias_H, weights_H):
    return pl.pallas_call(
        _pre_body,
        grid=(N // _BR,),
        in_specs=[
            pl.BlockSpec((1, N), lambda i: (0, 0)),
            pl.BlockSpec((N, D), lambda i: (0, 0)),
            pl.BlockSpec((1, N), lambda i: (0, 0)),
            pl.BlockSpec((_BR, N), lambda i: (i, 0)),
        ],
        out_specs=[
            pl.BlockSpec((1, N), lambda i: (0, 0)),
            pl.BlockSpec((1, N), lambda i: (0, 0)),
            pl.BlockSpec((N, D), lambda i: (0, 0)),
        ],
        out_shape=[
            jax.ShapeDtypeStruct((1, N), jnp.float32),
            jax.ShapeDtypeStruct((1, N), jnp.float32),
            jax.ShapeDtypeStruct((N, D), jnp.float32),
        ],
    )(state_H[None, :], state_K, bias_H[None, :], weights_H)


_mesh = plsc.VectorSubcoreMesh(core_axis_name="c", subcore_axis_name="s")


def _splat(ref1d, e):
    return plsc.load_gather(ref1d, [jnp.full((16,), e, jnp.int32)])


@functools.partial(
    pl.kernel,
    out_type=[
        jax.ShapeDtypeStruct((NC, N, D), jnp.float32),
        jax.ShapeDtypeStruct((NW, N), jnp.float32),
    ],
    mesh=_mesh,
    compiler_params=pltpu.CompilerParams(needs_layout_passes=False),
    scratch_types=[
        pltpu.VMEM((B,), jnp.int32),
        pltpu.VMEM((B,), jnp.int32),
        pltpu.VMEM((B,), jnp.int32),
        pltpu.VMEM((B,), jnp.int32),
        pltpu.VMEM((B,), jnp.int32),
        pltpu.VMEM((B,), jnp.int32),
        pltpu.VMEM((B,), jnp.int32),
        pltpu.VMEM((B,), jnp.int32),
        pltpu.VMEM((B,), jnp.float32),
        pltpu.VMEM((B,), jnp.float32),
        pltpu.VMEM((B,), jnp.float32),
        pltpu.VMEM((B,), jnp.float32),
        pltpu.VMEM((B, D), jnp.float32),
        pltpu.VMEM((B, D), jnp.float32),
        pltpu.VMEM((B, D), jnp.float32),
        pltpu.VMEM((B, D), jnp.float32),
        pltpu.VMEM((N,), jnp.float32),
        pltpu.VMEM((N,), jnp.float32),
        pltpu.VMEM_SHARED((N, D), jnp.float32),
        pltpu.SemaphoreType.DMA,
        pltpu.SemaphoreType.DMA,
        pltpu.SemaphoreType.DMA,
        pltpu.SemaphoreType.DMA,
        pltpu.SemaphoreType.DMA,
        pltpu.SemaphoreType.DMA,
    ],
)
def _edges(xn, g, ik0, ik1, wk, ih0, ih1, whk, whh, fko, fho,
           gi0, gj0, gi1, gj1, si0, sj0, si1, sj1,
           wa0, wa1, wb0, wb1,
           xi0, xj0, xi1, xj1, gall, fh, fk_acc,
           sg0, sg1, ss0, ss1, si0s, si1s):
    cid = lax.axis_index("c")
    sid = lax.axis_index("s")
    wid = cid * NS + sid
    z16 = jnp.zeros((16,), jnp.float32)
    lane0 = jnp.arange(16, dtype=jnp.int32) == 0
    GI = (gi0, gi1)
    GJ = (gj0, gj1)
    SI = (si0, si1)
    SJ = (sj0, sj1)
    WA = (wa0, wa1)
    WB = (wb0, wb1)
    XI = (xi0, xi1)
    XJ = (xj0, xj1)
    SG = (sg0, sg1)
    SS = (ss0, ss1)
    SM = (si0s, si1s)

    pltpu.sync_copy(g, gall)

    @plsc.parallel_loop(0, B)
    def _zrow(r):
        for k in range(D // 16):
            xi0[r, pl.ds(k * 16, 16)] = z16

    @plsc.parallel_loop(0, N // 16)
    def _zfh(i):
        fh[pl.ds(i * 16, 16)] = z16

    for r0 in range(0, RPT, B):
        pltpu.sync_copy(xi0, fk_acc.at[pl.ds(sid * RPT + r0, B)])
    plsc.subcore_barrier()

    def _phase(base0, C, i0_h, i1_h, wa_h, wb_h, hk):
        def _idx_dma(n, b):
            off = base0 + n * B
            pltpu.async_copy(i0_h.at[pl.ds(off, B)], GI[b], SM[b])
            pltpu.async_copy(i1_h.at[pl.ds(off, B)], GJ[b], SM[b])
            pltpu.async_copy(wa_h.at[pl.ds(off, B)], WA[b], SM[b])
            if hk:
                pltpu.async_copy(wb_h.at[pl.ds(off, B)], WB[b], SM[b])

        def _drain_idx(n, b):
            off = base0 + n * B
            pltpu.make_async_copy(i0_h.at[pl.ds(off, B)], GI[b], SM[b]).wait()
            pltpu.make_async_copy(i1_h.at[pl.ds(off, B)], GJ[b], SM[b]).wait()
            pltpu.make_async_copy(wa_h.at[pl.ds(off, B)], WA[b], SM[b]).wait()
            if hk:
                pltpu.make_async_copy(wb_h.at[pl.ds(off, B)], WB[b], SM[b]).wait()

        def _gather(b):
            pltpu.async_copy(xn.at[GI[b]], XI[b], SG[b])
            pltpu.async_copy(xn.at[GJ[b]], XJ[b], SG[b])

        def _drain_gather(b):
            pltpu.make_async_copy(xn.at[GI[b]], XI[b], SG[b]).wait()
            pltpu.make_async_copy(xn.at[GJ[b]], XJ[b], SG[b]).wait()

        def _scatter(b):
            pltpu.async_copy(XJ[b], fk_acc.at[SI[b]], SS[b], add=True)
            pltpu.async_copy(XI[b], fk_acc.at[SJ[b]], SS[b], add=True)

        def _drain_scatter(b):
            pltpu.make_async_copy(XJ[b], fk_acc.at[SI[b]], SS[b]).wait()
            pltpu.make_async_copy(XI[b], fk_acc.at[SJ[b]], SS[b]).wait()

        def _compute(b):
            xi_b, xj_b = XI[b], XJ[b]
            wa_b, wb_b = WA[b], WB[b]
            gi_b, gj_b = GI[b], GJ[b]

            @plsc.parallel_loop(0, B, unroll=16)
            def _edge(e):
                xiv = [xi_b[e, pl.ds(k * 16, 16)] for k in range(D // 16)]
                xjv = [xj_b[e, pl.ds(k * 16, 16)] for k in range(D // 16)]
                acc = xiv[0] * xjv[0]
                for k in range(1, D // 16):
                    acc = acc + xiv[k] * xjv[k]
                sv = jnp.broadcast_to(jnp.sum(acc), (16,))
                if hk:
                    iiv = _splat(gi_b, e)
                    jjv = _splat(gj_b, e)
                    giv = plsc.load_gather(gall, [iiv])
                    gjv = plsc.load_gather(gall, [jjv])
                    cv = _splat(wb_b, e) * sv
                    plsc.addupdate_scatter(fh, [iiv], cv * gjv, mask=lane0)
                    plsc.addupdate_scatter(fh, [jjv], cv * giv, mask=lane0)
                    qv = -(giv * gjv * _splat(wa_b, e))
                else:
                    qv = -_splat(wa_b, e) * sv
                for k in range(D // 16):
                    xj_b[e, pl.ds(k * 16, 16)] = qv * xjv[k]
                    xi_b[e, pl.ds(k * 16, 16)] = qv * xiv[k]

        def _save_idx(b):
            for k in range(B // 16):
                s = pl.ds(k * 16, 16)
                SI[b][s] = GI[b][s]
                SJ[b][s] = GJ[b][s]

        _idx_dma(0, 0)
        _idx_dma(1, 1)
        _drain_idx(0, 0)
        _gather(0)

        @pl.loop(0, C, step=2)
        def _loop(c):
            for b in (0, 1):
                o = 1 - b
                n = c + b
                if b == 0:
                    @pl.when(c > 0)
                    def _():
                        _drain_scatter(o)
                    _drain_idx(n + 1, o)
                    _gather(o)
                else:
                    _drain_scatter(o)

                    @pl.when(c < C - 2)
                    def _():
                        _drain_idx(n + 1, o)
                        _gather(o)
                _drain_gather(b)
                _save_idx(b)
                _compute(b)

                @pl.when(c < C - 2)
                def _():
                    _idx_dma(n + 2, b)
                _scatter(b)

        _drain_scatter((C - 1) % 2)

    _phase(wid * (CK * B), CK, ik0, ik1, wk, wk, False)
    _phase(wid * (CHK * B), CHK, ih0, ih1, whk, whh, True)
    plsc.subcore_barrier()

    pltpu.sync_copy(fk_acc.at[pl.ds(sid * RPT, RPT)],
                    fko.at[cid, pl.ds(sid * RPT, RPT)])
    pltpu.sync_copy(fh, fho.at[wid])


def _post_body(Xn_ref, fkp_ref, fhp_ref, h0_ref, fH_ref, fK_ref):
    fH_ref[...] = h0_ref[...] + jnp.sum(fhp_ref[...], axis=0, keepdims=True)
    X = Xn_ref[...]
    fKp = fkp_ref[0] + fkp_ref[1]
    proj = jnp.sum(X * fKp, axis=1, keepdims=True)
    fK_ref[...] = -fKp + X * proj


def _post(Xn, fkp, fhp, h0):
    return pl.pallas_call(
        _post_body,
        out_shape=[
            jax.ShapeDtypeStruct((1, N), jnp.float32),
            jax.ShapeDtypeStruct((N, D), jnp.float32),
        ],
    )(Xn, fkp, fhp, h0)


def kernel(t, state_H, state_K, ind_K, ind_HK, kappa_K, kappa_H, weights_H, bias_H, weights_HK, weights_K):
    g2, h0, Xn = _pre(state_H, state_K, bias_H, weights_H)
    g = g2.reshape(N)
    whh = weights_HK[:, 0] / kappa_H
    whk = weights_HK[:, 0] / kappa_K
    fkp, fhp = _edges(Xn, g, ind_K[:, 0], ind_K[:, 1], weights_K,
                      ind_HK[:, 0], ind_HK[:, 1], whk, whh)
    fH2, fK = _post(Xn, fkp, fhp, h0)
    return (fH2.reshape(N), fK)


# speedup vs baseline: 20.3894x; 1.0334x over previous
import functools

import jax
import jax.numpy as jnp
from jax import lax
from jax.experimental import pallas as pl
from jax.experimental.pallas import tpu as pltpu
from jax.experimental.pallas import tpu_sc as plsc

N = 4096
D = 128
EK = 262144
EHK = 131072
NC = 2
NS = 16
NW = NC * NS
B = 128
CK = EK // NW // B
CHK = EHK // NW // B
RPT = N // NS

_BR = 512


def _prea_body(sH_ref, sK_ref, g_ref, Xn_ref):
    g_ref[...] = jnp.tanh(sH_ref[...])
    X = sK_ref[...]
    norm = jnp.sqrt(jnp.sum(X * X, axis=1, keepdims=True))
    Xn_ref[...] = X / norm


def _prea(state_H, state_K):
    return pl.pallas_call(
        _prea_body,
        out_shape=[
            jax.ShapeDtypeStruct((1, N), jnp.float32),
            jax.ShapeDtypeStruct((N, D), jnp.float32),
        ],
    )(state_H[None, :], state_K)


def _preb_body(sH_ref, bH_ref, W_ref, h0_ref):
    step = pl.program_id(0)
    g = jnp.tanh(sH_ref[...])

    @pl.when(step == 0)
    def _init():
        h0_ref[...] = -sH_ref[...] + bH_ref[...]

    Wb = W_ref[...]
    wg = jnp.dot(Wb, g.T, preferred_element_type=jnp.float32)
    gs = jnp.tanh(sH_ref[:, pl.ds(step * _BR, _BR)])
    wtg = jnp.dot(gs, Wb, preferred_element_type=jnp.float32)
    h0_ref[...] += 0.5 * wtg
    h0_ref[:, pl.ds(step * _BR, _BR)] += 0.5 * wg.T


def _preb(state_H, bias_H, weights_H):
    return pl.pallas_call(
        _preb_body,
        grid=(N // _BR,),
        in_specs=[
            pl.BlockSpec((1, N), lambda i: (0, 0)),
            pl.BlockSpec((1, N), lambda i: (0, 0)),
            pl.BlockSpec((_BR, N), lambda i: (i, 0)),
        ],
        out_specs=[pl.BlockSpec((1, N), lambda i: (0, 0))],
        out_shape=[jax.ShapeDtypeStruct((1, N), jnp.float32)],
    )(state_H[None, :], bias_H[None, :], weights_H)


_mesh = plsc.VectorSubcoreMesh(core_axis_name="c", subcore_axis_name="s")


def _splat(ref1d, e):
    return plsc.load_gather(ref1d, [jnp.full((16,), e, jnp.int32)])


@functools.partial(
    pl.kernel,
    out_type=[
        jax.ShapeDtypeStruct((NC, N, D), jnp.float32),
        jax.ShapeDtypeStruct((NW, N), jnp.float32),
    ],
    mesh=_mesh,
    compiler_params=pltpu.CompilerParams(needs_layout_passes=False),
    scratch_types=[
        pltpu.VMEM((B,), jnp.int32),
        pltpu.VMEM((B,), jnp.int32),
        pltpu.VMEM((B,), jnp.int32),
        pltpu.VMEM((B,), jnp.int32),
        pltpu.VMEM((B,), jnp.int32),
        pltpu.VMEM((B,), jnp.int32),
        pltpu.VMEM((B,), jnp.int32),
        pltpu.VMEM((B,), jnp.int32),
        pltpu.VMEM((B,), jnp.float32),
        pltpu.VMEM((B,), jnp.float32),
        pltpu.VMEM((B,), jnp.float32),
        pltpu.VMEM((B,), jnp.float32),
        pltpu.VMEM((B, D), jnp.float32),
        pltpu.VMEM((B, D), jnp.float32),
        pltpu.VMEM((B, D), jnp.float32),
        pltpu.VMEM((B, D), jnp.float32),
        pltpu.VMEM((N,), jnp.float32),
        pltpu.VMEM((N,), jnp.float32),
        pltpu.VMEM_SHARED((N, D), jnp.float32),
        pltpu.SemaphoreType.DMA,
        pltpu.SemaphoreType.DMA,
        pltpu.SemaphoreType.DMA,
        pltpu.SemaphoreType.DMA,
        pltpu.SemaphoreType.DMA,
        pltpu.SemaphoreType.DMA,
    ],
)
def _edges(xn, g, ik0, ik1, wk, ih0, ih1, whk, whh, fko, fho,
           gi0, gj0, gi1, gj1, si0, sj0, si1, sj1,
           wa0, wa1, wb0, wb1,
           xi0, xj0, xi1, xj1, gall, fh, fk_acc,
           sg0, sg1, ss0, ss1, si0s, si1s):
    cid = lax.axis_index("c")
    sid = lax.axis_index("s")
    wid = cid * NS + sid
    z16 = jnp.zeros((16,), jnp.float32)
    lane0 = jnp.arange(16, dtype=jnp.int32) == 0
    GI = (gi0, gi1)
    GJ = (gj0, gj1)
    SI = (si0, si1)
    SJ = (sj0, sj1)
    WA = (wa0, wa1)
    WB = (wb0, wb1)
    XI = (xi0, xi1)
    XJ = (xj0, xj1)
    SG = (sg0, sg1)
    SS = (ss0, ss1)
    SM = (si0s, si1s)

    pltpu.sync_copy(g, gall)

    @plsc.parallel_loop(0, B)
    def _zrow(r):
        for k in range(D // 16):
            xi0[r, pl.ds(k * 16, 16)] = z16

    @plsc.parallel_loop(0, N // 16)
    def _zfh(i):
        fh[pl.ds(i * 16, 16)] = z16

    for r0 in range(0, RPT, B):
        pltpu.sync_copy(xi0, fk_acc.at[pl.ds(sid * RPT + r0, B)])
    plsc.subcore_barrier()

    def _phase(base0, C, i0_h, i1_h, wa_h, wb_h, hk):
        def _idx_dma(n, b):
            off = base0 + n * B
            pltpu.async_copy(i0_h.at[pl.ds(off, B)], GI[b], SM[b])
            pltpu.async_copy(i1_h.at[pl.ds(off, B)], GJ[b], SM[b])
            pltpu.async_copy(wa_h.at[pl.ds(off, B)], WA[b], SM[b])
            if hk:
                pltpu.async_copy(wb_h.at[pl.ds(off, B)], WB[b], SM[b])

        def _drain_idx(n, b):
            off = base0 + n * B
            pltpu.make_async_copy(i0_h.at[pl.ds(off, B)], GI[b], SM[b]).wait()
            pltpu.make_async_copy(i1_h.at[pl.ds(off, B)], GJ[b], SM[b]).wait()
            pltpu.make_async_copy(wa_h.at[pl.ds(off, B)], WA[b], SM[b]).wait()
            if hk:
                pltpu.make_async_copy(wb_h.at[pl.ds(off, B)], WB[b], SM[b]).wait()

        def _gather(b):
            pltpu.async_copy(xn.at[GI[b]], XI[b], SG[b])
            pltpu.async_copy(xn.at[GJ[b]], XJ[b], SG[b])

        def _drain_gather(b):
            pltpu.make_async_copy(xn.at[GI[b]], XI[b], SG[b]).wait()
            pltpu.make_async_copy(xn.at[GJ[b]], XJ[b], SG[b]).wait()

        def _scatter(b):
            pltpu.async_copy(XJ[b], fk_acc.at[SI[b]], SS[b], add=True)
            pltpu.async_copy(XI[b], fk_acc.at[SJ[b]], SS[b], add=True)

        def _drain_scatter(b):
            pltpu.make_async_copy(XJ[b], fk_acc.at[SI[b]], SS[b]).wait()
            pltpu.make_async_copy(XI[b], fk_acc.at[SJ[b]], SS[b]).wait()

        def _compute(b):
            xi_b, xj_b = XI[b], XJ[b]
            wa_b, wb_b = WA[b], WB[b]
            gi_b, gj_b = GI[b], GJ[b]

            @plsc.parallel_loop(0, B, unroll=8)
            def _edge(e):
                xiv = [xi_b[e, pl.ds(k * 16, 16)] for k in range(D // 16)]
                xjv = [xj_b[e, pl.ds(k * 16, 16)] for k in range(D // 16)]
                acc = xiv[0] * xjv[0]
                for k in range(1, D // 16):
                    acc = acc + xiv[k] * xjv[k]
                sv = jnp.broadcast_to(jnp.sum(acc), (16,))
                if hk:
                    iiv = _splat(gi_b, e)
                    jjv = _splat(gj_b, e)
                    giv = plsc.load_gather(gall, [iiv])
                    gjv = plsc.load_gather(gall, [jjv])
                    cv = _splat(wb_b, e) * sv
                    plsc.addupdate_scatter(fh, [iiv], cv * gjv, mask=lane0)
                    plsc.addupdate_scatter(fh, [jjv], cv * giv, mask=lane0)
                    qv = -(giv * gjv * _splat(wa_b, e))
                else:
                    qv = -_splat(wa_b, e) * sv
                for k in range(D // 16):
                    xj_b[e, pl.ds(k * 16, 16)] = qv * xjv[k]
                    xi_b[e, pl.ds(k * 16, 16)] = qv * xiv[k]

        def _save_idx(b):
            for k in range(B // 16):
                s = pl.ds(k * 16, 16)
                SI[b][s] = GI[b][s]
                SJ[b][s] = GJ[b][s]

        _idx_dma(0, 0)
        _idx_dma(1, 1)
        _drain_idx(0, 0)
        _gather(0)

        @pl.loop(0, C, step=2)
        def _loop(c):
            for b in (0, 1):
                o = 1 - b
                n = c + b
                if b == 0:
                    @pl.when(c > 0)
                    def _():
                        _drain_scatter(o)
                    _drain_idx(n + 1, o)
                    _gather(o)
                else:
                    _drain_scatter(o)

                    @pl.when(c < C - 2)
                    def _():
                        _drain_idx(n + 1, o)
                        _gather(o)
                _drain_gather(b)
                _save_idx(b)
                _compute(b)

                @pl.when(c < C - 2)
                def _():
                    _idx_dma(n + 2, b)
                _scatter(b)

        _drain_scatter((C - 1) % 2)

    _phase(wid * (CK * B), CK, ik0, ik1, wk, wk, False)
    _phase(wid * (CHK * B), CHK, ih0, ih1, whk, whh, True)
    plsc.subcore_barrier()

    pltpu.sync_copy(fk_acc.at[pl.ds(sid * RPT, RPT)],
                    fko.at[cid, pl.ds(sid * RPT, RPT)])
    pltpu.sync_copy(fh, fho.at[wid])


def _post_body(Xn_ref, fkp_ref, fhp_ref, h0_ref, fH_ref, fK_ref):
    fH_ref[...] = h0_ref[...] + jnp.sum(fhp_ref[...], axis=0, keepdims=True)
    X = Xn_ref[...]
    fKp = fkp_ref[0] + fkp_ref[1]
    proj = jnp.sum(X * fKp, axis=1, keepdims=True)
    fK_ref[...] = -fKp + X * proj


def _post(Xn, fkp, fhp, h0):
    return pl.pallas_call(
        _post_body,
        out_shape=[
            jax.ShapeDtypeStruct((1, N), jnp.float32),
            jax.ShapeDtypeStruct((N, D), jnp.float32),
        ],
    )(Xn, fkp, fhp, h0)


def kernel(t, state_H, state_K, ind_K, ind_HK, kappa_K, kappa_H, weights_H, bias_H, weights_HK, weights_K):
    g2, Xn = _prea(state_H, state_K)
    (h0,) = _preb(state_H, bias_H, weights_H)
    g = g2.reshape(N)
    whh = weights_HK[:, 0] / kappa_H
    whk = weights_HK[:, 0] / kappa_K
    fkp, fhp = _edges(Xn, g, ind_K[:, 0], ind_K[:, 1], weights_K,
                      ind_HK[:, 0], ind_HK[:, 1], whk, whh)
    fH2, fK = _post(Xn, fkp, fhp, h0)
    return (fH2.reshape(N), fK)
